# baseline (speedup 1.0000x reference)
import jax
import jax.numpy as jnp
import numpy as np
from jax.experimental import pallas as pl

_T = 2048
_F = _T // 2 + 1
_FPAD = 1152
_K = 4
_PMAX = 64
_PMIN = 32
_NP = _PMAX - _PMIN + 1
_FOLDPAD = 2176


def _build_wdft() -> np.ndarray:
    t = np.arange(_T, dtype=np.float64)
    f = np.arange(_F, dtype=np.float64)
    ang = (2.0 * np.pi / _T) * np.outer(t, f)
    w = np.zeros((_T, 2 * _FPAD), dtype=np.float32)
    w[:, :_F] = np.cos(ang).astype(np.float32)
    w[:, _FPAD:_FPAD + _F] = -np.sin(ang).astype(np.float32)
    w[:, 0] = 0.0
    w[:, _FPAD] = 0.0
    return w


def _build_wfold() -> np.ndarray:
    w = np.zeros((_T, _FOLDPAD), dtype=np.float32)
    for j in range(_NP):
        p = _PMIN + j
        cycles = _T // p
        start = _T - cycles * p
        tt = np.arange(start, _T)
        w[tt, j * _PMAX + ((tt - start) % p)] = 1.0
    return w


_WDFT = _build_wdft()
_WFOLD = _build_wfold().astype(jnp.bfloat16)
_INV_CYCLES = [1.0 / (_T // (_PMIN + j)) for j in range(_NP)]


_DNT = (((0,), (0,)), ((), ()))


def _dft_body(x_ref, w_ref, y_ref):
    y_ref[0] = jax.lax.dot_general(
        x_ref[0], w_ref[...], _DNT,
        preferred_element_type=jnp.float32,
        precision=jax.lax.Precision.HIGHEST,
    )


def _fold_body(xh_ref, xl_ref, w_ref, y_ref):
    y_ref[0] = (
        jax.lax.dot_general(xh_ref[0], w_ref[...], _DNT,
                            preferred_element_type=jnp.float32)
        + jax.lax.dot_general(xl_ref[0], w_ref[...], _DNT,
                              preferred_element_type=jnp.float32)
    )


def _select_body(c_ref, s_ref, fold_ref, o_ref):
    rows = c_ref.shape[0]
    c = c_ref[...]
    s = s_ref[...]
    mag2 = c * c + s * s
    lane = jax.lax.broadcasted_iota(jnp.int32, (rows, _FPAD), 1)
    kidxs = []
    for _ in range(_K):
        m = jnp.max(mag2, axis=1, keepdims=True)
        hit = mag2 == m
        idx = jnp.min(jnp.where(hit, lane, _FPAD), axis=1, keepdims=True)
        kidxs.append(idx)
        mag2 = jnp.where(lane == idx, jnp.float32(-1.0), mag2)
    for k in range(_K):
        kidx = jnp.maximum(kidxs[k], 1).astype(jnp.float32)
        p = jnp.clip(jnp.floor(jnp.float32(_T) / kidx), _PMIN, _PMAX)
        pidx = p.astype(jnp.int32) - _PMIN
        acc = jnp.zeros((rows, _PMAX), dtype=jnp.float32)
        for j in range(_NP):
            fold_j = fold_ref[:, j * _PMAX:(j + 1) * _PMAX]
            acc = acc + jnp.where(pidx == j,
                                  fold_j * jnp.float32(_INV_CYCLES[j]),
                                  jnp.float32(0.0))
        o_ref[:, k * _PMAX:(k + 1) * _PMAX] = acc


@jax.jit
def kernel(x):
    B, T, N = x.shape
    BN = B * N
    xh = x.astype(jnp.bfloat16)
    xl = (x - xh.astype(jnp.float32)).astype(jnp.bfloat16)
    wdft = jnp.asarray(_WDFT)
    wfold = jnp.asarray(_WFOLD)

    cb = 768
    ydft = pl.pallas_call(
        _dft_body,
        grid=((2 * _FPAD) // cb, B),
        in_specs=[
            pl.BlockSpec((1, _T, N), lambda j, b: (b, 0, 0)),
            pl.BlockSpec((_T, cb), lambda j, b: (0, j)),
        ],
        out_specs=pl.BlockSpec((1, N, cb), lambda j, b: (b, 0, j)),
        out_shape=jax.ShapeDtypeStruct((B, N, 2 * _FPAD), jnp.float32),
    )(x, wdft).reshape(BN, 2 * _FPAD)

    yfold = pl.pallas_call(
        _fold_body,
        grid=(B,),
        in_specs=[
            pl.BlockSpec((1, _T, N), lambda b: (b, 0, 0)),
            pl.BlockSpec((1, _T, N), lambda b: (b, 0, 0)),
            pl.BlockSpec((_T, _FOLDPAD), lambda b: (0, 0)),
        ],
        out_specs=pl.BlockSpec((1, N, _FOLDPAD), lambda b: (b, 0, 0)),
        out_shape=jax.ShapeDtypeStruct((B, N, _FOLDPAD), jnp.float32),
    )(xh, xl, wfold).reshape(BN, _FOLDPAD)

    rb2 = 128
    out = pl.pallas_call(
        _select_body,
        grid=(BN // rb2,),
        in_specs=[
            pl.BlockSpec((rb2, _FPAD), lambda i: (i, 0)),
            pl.BlockSpec((rb2, _FPAD), lambda i: (i, 1)),
            pl.BlockSpec((rb2, _FOLDPAD), lambda i: (i, 0)),
        ],
        out_specs=pl.BlockSpec((rb2, _K * _PMAX), lambda i: (i, 0)),
        out_shape=jax.ShapeDtypeStruct((BN, _K * _PMAX), jnp.float32),
    )(ydft, ydft, yfold)

    return out.reshape(B, N, _K, _PMAX).transpose(0, 2, 3, 1)


# speedup vs baseline: 2.1410x; 2.1410x over previous
import jax
import jax.numpy as jnp
import numpy as np
from jax.experimental import pallas as pl

_T = 2048
_F = _T // 2 + 1
_FPAD = 1152
_K = 4
_PMAX = 64
_PMIN = 32
_NP = _PMAX - _PMIN + 1
_FOLDPAD = 2176


def _build_wdft() -> np.ndarray:
    t = np.arange(_T, dtype=np.float64)
    f = np.arange(_F, dtype=np.float64)
    ang = (2.0 * np.pi / _T) * np.outer(t, f)
    w = np.zeros((_T, 2 * _FPAD), dtype=np.float32)
    w[:, :_F] = np.cos(ang).astype(np.float32)
    w[:, _FPAD:_FPAD + _F] = -np.sin(ang).astype(np.float32)
    w[:, 0] = 0.0
    w[:, _FPAD] = 0.0
    return w


def _build_wfold() -> np.ndarray:
    w = np.zeros((_T, _FOLDPAD), dtype=np.float32)
    for j in range(_NP):
        p = _PMIN + j
        cycles = _T // p
        start = _T - cycles * p
        tt = np.arange(start, _T)
        w[tt, j * _PMAX + ((tt - start) % p)] = 1.0
    return w


_WDFT = _build_wdft()
_WFOLD = _build_wfold().astype(jnp.bfloat16)
_INV_CYCLES = [1.0 / (_T // (_PMIN + j)) for j in range(_NP)]


_DN = (((1,), (0,)), ((), ()))


def _seqs_block(x_ref):
    nb = x_ref.shape[0]
    return jnp.concatenate(
        [jnp.transpose(x_ref[b], (1, 0)) for b in range(nb)], axis=0)


def _dft_body(x_ref, w_ref, y_ref):
    seqs = _seqs_block(x_ref)
    y_ref[...] = jax.lax.dot_general(
        seqs, w_ref[...], _DN,
        preferred_element_type=jnp.float32,
        precision=jax.lax.Precision.HIGHEST,
    )


def _fold_body(x_ref, w_ref, y_ref):
    seqs = _seqs_block(x_ref)
    xh = seqs.astype(jnp.bfloat16)
    xl = (seqs - xh.astype(jnp.float32)).astype(jnp.bfloat16)
    y_ref[...] = (
        jax.lax.dot_general(xh, w_ref[...], _DN,
                            preferred_element_type=jnp.float32)
        + jax.lax.dot_general(xl, w_ref[...], _DN,
                              preferred_element_type=jnp.float32)
    )


def _select_body(c_ref, s_ref, fold_ref, o_ref):
    rows = c_ref.shape[0]
    c = c_ref[...]
    s = s_ref[...]
    mag2 = c * c + s * s
    lane = jax.lax.broadcasted_iota(jnp.int32, (rows, _FPAD), 1)
    kidxs = []
    for _ in range(_K):
        m = jnp.max(mag2, axis=1, keepdims=True)
        hit = mag2 == m
        idx = jnp.min(jnp.where(hit, lane, _FPAD), axis=1, keepdims=True)
        kidxs.append(idx)
        mag2 = jnp.where(lane == idx, jnp.float32(-1.0), mag2)
    for k in range(_K):
        kidx = jnp.maximum(kidxs[k], 1).astype(jnp.float32)
        p = jnp.clip(jnp.floor(jnp.float32(_T) / kidx), _PMIN, _PMAX)
        pidx = p.astype(jnp.int32) - _PMIN
        acc = jnp.zeros((rows, _PMAX), dtype=jnp.float32)
        for j in range(_NP):
            fold_j = fold_ref[:, j * _PMAX:(j + 1) * _PMAX]
            acc = acc + jnp.where(pidx == j,
                                  fold_j * jnp.float32(_INV_CYCLES[j]),
                                  jnp.float32(0.0))
        o_ref[:, k * _PMAX:(k + 1) * _PMAX] = acc


@jax.jit
def kernel(x):
    B, T, N = x.shape
    BN = B * N
    wdft = jnp.asarray(_WDFT)
    wfold = jnp.asarray(_WFOLD)

    nb, cb = 4, 768
    ydft = pl.pallas_call(
        _dft_body,
        grid=(B // nb, (2 * _FPAD) // cb),
        in_specs=[
            pl.BlockSpec((nb, _T, N), lambda i, j: (i, 0, 0)),
            pl.BlockSpec((_T, cb), lambda i, j: (0, j)),
        ],
        out_specs=pl.BlockSpec((nb * N, cb), lambda i, j: (i, j)),
        out_shape=jax.ShapeDtypeStruct((BN, 2 * _FPAD), jnp.float32),
    )(x, wdft)

    yfold = pl.pallas_call(
        _fold_body,
        grid=(B // nb,),
        in_specs=[
            pl.BlockSpec((nb, _T, N), lambda i: (i, 0, 0)),
            pl.BlockSpec((_T, _FOLDPAD), lambda i: (0, 0)),
        ],
        out_specs=pl.BlockSpec((nb * N, _FOLDPAD), lambda i: (i, 0)),
        out_shape=jax.ShapeDtypeStruct((BN, _FOLDPAD), jnp.float32),
    )(x, wfold)

    rb2 = 128
    out = pl.pallas_call(
        _select_body,
        grid=(BN // rb2,),
        in_specs=[
            pl.BlockSpec((rb2, _FPAD), lambda i: (i, 0)),
            pl.BlockSpec((rb2, _FPAD), lambda i: (i, 1)),
            pl.BlockSpec((rb2, _FOLDPAD), lambda i: (i, 0)),
        ],
        out_specs=pl.BlockSpec((rb2, _K * _PMAX), lambda i: (i, 0)),
        out_shape=jax.ShapeDtypeStruct((BN, _K * _PMAX), jnp.float32),
    )(ydft, ydft, yfold)

    return out.reshape(B, N, _K, _PMAX).transpose(0, 2, 3, 1)


# speedup vs baseline: 2.5143x; 1.1744x over previous
import jax
import jax.numpy as jnp
import numpy as np
from jax.experimental import pallas as pl

_T = 2048
_F = _T // 2 + 1
_FPAD = 1152
_K = 4
_PMAX = 64
_PMIN = 32
_NP = _PMAX - _PMIN + 1
_FOLDPAD = 2176


def _build_wdft() -> np.ndarray:
    t = np.arange(_T, dtype=np.float64)
    f = np.arange(_F, dtype=np.float64)
    ang = (2.0 * np.pi / _T) * np.outer(t, f)
    w = np.zeros((_T, 2 * _FPAD), dtype=np.float32)
    w[:, :_F] = np.cos(ang).astype(np.float32)
    w[:, _FPAD:_FPAD + _F] = -np.sin(ang).astype(np.float32)
    w[:, 0] = 0.0
    w[:, _FPAD] = 0.0
    return w


def _build_wfold() -> np.ndarray:
    w = np.zeros((_T, _FOLDPAD), dtype=np.float32)
    for j in range(_NP):
        p = _PMIN + j
        cycles = _T // p
        start = _T - cycles * p
        tt = np.arange(start, _T)
        w[tt, j * _PMAX + ((tt - start) % p)] = 1.0
    return w


_WDFT = _build_wdft()
_WFOLD = _build_wfold().astype(jnp.bfloat16)
_INV_CYCLES = [1.0 / (_T // (_PMIN + j)) for j in range(_NP)]


_DN = (((1,), (0,)), ((), ()))


def _dft_body(x_ref, w_ref, y_ref):
    y_ref[...] = jax.lax.dot_general(
        x_ref[...], w_ref[...], _DN,
        preferred_element_type=jnp.float32,
        precision=jax.lax.Precision.HIGHEST,
    )


def _fold_body(x_ref, w_ref, y_ref):
    seqs = x_ref[...]
    xh = seqs.astype(jnp.bfloat16)
    xl = (seqs - xh.astype(jnp.float32)).astype(jnp.bfloat16)
    y_ref[...] = (
        jax.lax.dot_general(xh, w_ref[...], _DN,
                            preferred_element_type=jnp.float32)
        + jax.lax.dot_general(xl, w_ref[...], _DN,
                              preferred_element_type=jnp.float32)
    )


def _select_body(c_ref, s_ref, fold_ref, o_ref):
    rows = c_ref.shape[0]
    c = c_ref[...]
    s = s_ref[...]
    mag2 = c * c + s * s
    lane = jax.lax.broadcasted_iota(jnp.int32, (rows, _FPAD), 1)
    kidxs = []
    for _ in range(_K):
        m = jnp.max(mag2, axis=1, keepdims=True)
        hit = mag2 == m
        idx = jnp.min(jnp.where(hit, lane, _FPAD), axis=1, keepdims=True)
        kidxs.append(idx)
        mag2 = jnp.where(lane == idx, jnp.float32(-1.0), mag2)
    for k in range(_K):
        kidx = jnp.maximum(kidxs[k], 1).astype(jnp.float32)
        p = jnp.clip(jnp.floor(jnp.float32(_T) / kidx), _PMIN, _PMAX)
        pidx = p.astype(jnp.int32) - _PMIN
        acc = jnp.zeros((rows, _PMAX), dtype=jnp.float32)
        for j in range(_NP):
            fold_j = fold_ref[:, j * _PMAX:(j + 1) * _PMAX]
            acc = acc + jnp.where(pidx == j,
                                  fold_j * jnp.float32(_INV_CYCLES[j]),
                                  jnp.float32(0.0))
        o_ref[:, k * _PMAX:(k + 1) * _PMAX] = acc


@jax.jit
def kernel(x):
    B, T, N = x.shape
    BN = B * N
    seqs = jnp.transpose(x, (0, 2, 1)).reshape(BN, T)
    wdft = jnp.asarray(_WDFT)
    wfold = jnp.asarray(_WFOLD)

    rb = 256
    ydft = pl.pallas_call(
        _dft_body,
        grid=(BN // rb,),
        in_specs=[
            pl.BlockSpec((rb, _T), lambda i: (i, 0)),
            pl.BlockSpec((_T, 2 * _FPAD), lambda i: (0, 0)),
        ],
        out_specs=pl.BlockSpec((rb, 2 * _FPAD), lambda i: (i, 0)),
        out_shape=jax.ShapeDtypeStruct((BN, 2 * _FPAD), jnp.float32),
    )(seqs, wdft)

    yfold = pl.pallas_call(
        _fold_body,
        grid=(BN // rb,),
        in_specs=[
            pl.BlockSpec((rb, _T), lambda i: (i, 0)),
            pl.BlockSpec((_T, _FOLDPAD), lambda i: (0, 0)),
        ],
        out_specs=pl.BlockSpec((rb, _FOLDPAD), lambda i: (i, 0)),
        out_shape=jax.ShapeDtypeStruct((BN, _FOLDPAD), jnp.float32),
    )(seqs, wfold)

    rb2 = 128
    out = pl.pallas_call(
        _select_body,
        grid=(BN // rb2,),
        in_specs=[
            pl.BlockSpec((rb2, _FPAD), lambda i: (i, 0)),
            pl.BlockSpec((rb2, _FPAD), lambda i: (i, 1)),
            pl.BlockSpec((rb2, _FOLDPAD), lambda i: (i, 0)),
        ],
        out_specs=pl.BlockSpec((rb2, _K * _PMAX), lambda i: (i, 0)),
        out_shape=jax.ShapeDtypeStruct((BN, _K * _PMAX), jnp.float32),
    )(ydft, ydft, yfold)

    return out.reshape(B, N, _K, _PMAX).transpose(0, 2, 3, 1)
